# baseline (speedup 1.0000x reference)
import math

import jax
import jax.numpy as jnp
import numpy as np
from jax.experimental import pallas as pl

_B, _C1, _H, _W = 4, 192, 224, 224
_P = 4
_GH, _GW = _H // _P, _W // _P
_NP = _GH * _GW
_K = min(_NP, max(1, int(math.ceil(_NP * 0.25))))
_HID = 96
_RHID = 48
_EPS = 1e-5
_ROWS = 4
_BY = _ROWS * _P
_NB = _GH // _ROWS
_L = 2 * _GW

_f32 = jnp.float32


def _build_consts():
    s0 = np.zeros((_W, _L), np.float32)
    s1 = np.zeros((_W, _L), np.float32)
    for p in range(_GW):
        for ox in range(2):
            s0[4 * p + 2 * ox + 0, 2 * p + ox] = 1.0
            s1[4 * p + 2 * ox + 1, 2 * p + ox] = 1.0
    rp = np.zeros((_L, _L), np.float32)
    rm = np.zeros((_L, _L), np.float32)
    for j in range(_GW):
        rp[2 * j, 2 * j + 1] = 1.0
        rm[2 * j + 1, 2 * j] = 1.0
    spool = np.zeros((_W, _GW), np.float32)
    for p in range(_GW):
        for px in range(4):
            spool[4 * p + px, p] = 1.0
    return s0, s1, rp, rm, spool


_S0, _S1, _RP, _RM, _SPOOL = _build_consts()


def _dotx(a, b):
    return jnp.dot(a, b, precision=jax.lax.Precision.HIGHEST,
                   preferred_element_type=_f32)


def _dotd(a, b):
    return jnp.dot(a.astype(jnp.bfloat16), b.astype(jnp.bfloat16),
                   preferred_element_type=_f32)


def _silu(v):
    return v * jax.nn.sigmoid(v)


def _pool_route_body(x_ref, spool_ref, w1_ref, g1_ref, b1_ref, w2_ref,
                     b2_ref, out_ref):
    xb = x_ref[0]
    spool = spool_ref[...]
    rows = []
    for r2 in range(_ROWS):
        xs = jax.lax.slice(xb, (0, r2 * _P, 0), (_C1, r2 * _P + _P, _W))
        ysum = jnp.sum(xs, axis=1)
        pooled = _dotx(ysum, spool) * _f32(1.0 / 16.0)
        s1 = _dotd(w1_ref[...], pooled)
        s1 = s1 * g1_ref[0][:, None] + b1_ref[0][:, None]
        s1 = _silu(s1)
        lg = _dotd(w2_ref[...], s1)
        rows.append(lg + b2_ref[0][:, None])
    out_ref[0, 0] = jnp.concatenate(rows, axis=0)


def _threshold_body(lg_ref, out_ref):
    s = lg_ref[...]
    lo = jnp.min(s, axis=1, keepdims=True) - 1.0
    hi = jnp.max(s, axis=1, keepdims=True) + 1.0
    kf = _f32(_K)

    def step(_, carry):
        lo, hi = carry
        mid = (lo + hi) * 0.5
        cnt = jnp.sum((s >= mid).astype(_f32), axis=1, keepdims=True)
        pred = cnt >= kf
        return jnp.where(pred, mid, lo), jnp.where(pred, hi, mid)

    lo, hi = jax.lax.fori_loop(0, 60, step, (lo, hi))
    mask = (s >= lo).astype(_f32)
    out_ref[...] = jax.nn.sigmoid(s) * mask


def _mov(a, b):
    hi = a.astype(jnp.bfloat16)
    lo = (a - hi.astype(_f32)).astype(jnp.bfloat16)
    bb = b.astype(jnp.bfloat16)
    return (jnp.dot(hi, bb, preferred_element_type=_f32) +
            jnp.dot(lo, bb, preferred_element_type=_f32))


def _expert_body(x_ref, sc_ref, w1cc_ref, g1d_ref, b1d_ref, dw9_ref, dg_ref,
                 db_ref, w2e_ref, w2o_ref, g2m_ref, be_ref, s0_ref, s1_ref,
                 s0t_ref, s1t_ref, rp_ref, rm_ref, out_ref):
    xb = x_ref[0]
    s0 = s0_ref[...]
    s1 = s1_ref[...]
    s0t = s0t_ref[...]
    s1t = s1t_ref[...]
    rp = rp_ref[...]
    rm = rm_ref[...]
    w1cc = w1cc_ref[...]
    w2e = w2e_ref[...]
    w2o = w2o_ref[...]
    g2m = g2m_ref[...]
    be = be_ref[...]
    g1d = g1d_ref[0][:, None]
    b1d = b1d_ref[0][:, None]
    dg = dg_ref[0][:, None]
    db = db_ref[0][:, None]
    for r2 in range(_ROWS):
        xs = jax.lax.slice(xb, (0, r2 * _P, 0), (_C1, r2 * _P + _P, _W))
        sc = sc_ref[0, 0, r2]
        gs = []
        for oy in range(2):
            y2 = jnp.concatenate(
                [xs[:, 2 * oy, :], xs[:, 2 * oy + 1, :]], axis=0)
            gs.append(_dotd(w1cc, y2))
        ga = jnp.concatenate([gs[0][:_HID], gs[1][:_HID]], axis=0)
        gb = jnp.concatenate([gs[0][_HID:], gs[1][_HID:]], axis=0)
        tl = _mov(ga, s0) + _mov(gb, s1)
        tl = _silu(tl * g1d + b1d)
        tp = _mov(tl, rp)
        tm = _mov(tl, rm)
        us = []
        for oy_ in range(2):
            acc = None
            for oy in range(2):
                ky = 1 + oy - oy_
                k0 = dw9_ref[ky * 3 + 0][:, None]
                k1 = dw9_ref[ky * 3 + 1][:, None]
                k2 = dw9_ref[ky * 3 + 2][:, None]
                sl0, sl1 = oy * _HID, (oy + 1) * _HID
                term = (tl[sl0:sl1] * k1 + tp[sl0:sl1] * k0 +
                        tm[sl0:sl1] * k2)
                acc = term if acc is None else acc + term
            us.append(_silu(acc * dg + db))
        u2 = jnp.concatenate(us, axis=0)
        ue0 = _mov(u2, s0t)
        ue1 = _mov(u2, s1t)
        for oy_ in range(2):
            sl0, sl1 = oy_ * _HID, (oy_ + 1) * _HID
            r = (_dotd(w2e, ue0[sl0:sl1]) +
                 _dotd(w2o, ue1[sl0:sl1]))
            r = (r * g2m + be) * sc[None, :]
            out_ref[0, :, r2 * _P + oy_ * 2 + 0, :] = r[:_C1]
            out_ref[0, :, r2 * _P + oy_ * 2 + 1, :] = r[_C1:]


@jax.jit
def kernel(x, rw1, r_g, r_b, rw2, rb2, ew1, e1_g, e1_b, dw, d_g, d_b, ew2,
           e2_g, e2_b):
    sq = jnp.sqrt(_f32(1.0) + _f32(_EPS))
    r_gs = (r_g / sq).reshape(1, _RHID)
    e1sc = (e1_g / sq).reshape(1, _HID)
    dsc = (d_g / sq).reshape(1, _HID)
    e2v = (e2_g / sq)

    rw1r = rw1.reshape(_RHID, _C1)
    rw2r = rw2.reshape(1, _RHID)
    ew1r = ew1.reshape(_HID, 4 * _C1)
    ew2r = ew2.reshape(4 * _C1, _HID)

    w1r = ew1r.reshape(_HID, _C1, 2, 2)
    w1c0 = w1r[:, :, :, 0].transpose(0, 2, 1).reshape(_HID, 2 * _C1)
    w1c1 = w1r[:, :, :, 1].transpose(0, 2, 1).reshape(_HID, 2 * _C1)
    w1cc = jnp.concatenate([w1c0, w1c1], axis=0)
    w2r = ew2r.reshape(_C1, 2, 2, _HID)
    w2e = w2r[:, :, 0, :].transpose(1, 0, 2).reshape(2 * _C1, _HID)
    w2o = w2r[:, :, 1, :].transpose(1, 0, 2).reshape(2 * _C1, _HID)
    lane_dx = (np.arange(_W) % 2).astype(np.float32)
    m_e = jnp.asarray((1.0 - lane_dx)[None, :])
    m_o = jnp.asarray(lane_dx[None, :])
    ev = e2v.reshape(_C1, 2, 2)
    eb = e2_b.reshape(_C1, 2, 2)
    col = lambda a, dx: a[:, :, dx].transpose(1, 0).reshape(2 * _C1, 1)
    g2m = col(ev, 0) * m_e + col(ev, 1) * m_o
    be = col(eb, 0) * m_e + col(eb, 1) * m_o
    g1d = jnp.concatenate([e1sc, e1sc], axis=1)
    b1d = jnp.concatenate([e1_b.reshape(1, _HID)] * 2, axis=1)
    dw9 = dw.reshape(_HID, 3, 3).transpose(1, 2, 0).reshape(9, _HID)

    grid = (_B, _NB)
    x_spec = pl.BlockSpec((1, _C1, _BY, _W), lambda b, r: (b, 0, r, 0))
    lg_spec = pl.BlockSpec((1, 1, _ROWS, _GW), lambda b, r: (b, r, 0, 0))
    sc_spec = pl.BlockSpec((1, 1, _ROWS, _W), lambda b, r: (b, r, 0, 0))
    full = lambda shape: pl.BlockSpec(shape, lambda b, r: (0,) * len(shape))

    logits = pl.pallas_call(
        _pool_route_body,
        grid=grid,
        in_specs=[
            x_spec,
            full((_W, _GW)),
            full((_RHID, _C1)),
            full((1, _RHID)),
            full((1, _RHID)),
            full((1, _RHID)),
            full((1, 1)),
        ],
        out_specs=lg_spec,
        out_shape=jax.ShapeDtypeStruct((_B, _NB, _ROWS, _GW), _f32),
    )(x, jnp.asarray(_SPOOL), rw1r, r_gs, r_b.reshape(1, _RHID), rw2r,
      rb2.reshape(1, 1))

    scale = pl.pallas_call(
        _threshold_body,
        out_shape=jax.ShapeDtypeStruct((_B, _NP), _f32),
    )(logits.reshape(_B, _NP))
    sc2 = jnp.repeat(scale.reshape(_B, _GH, _GW), 4, axis=2)
    sc2 = sc2.reshape(_B, _NB, _ROWS, _W)

    delta = pl.pallas_call(
        _expert_body,
        grid=grid,
        in_specs=[
            x_spec,
            sc_spec,
            full((2 * _HID, 2 * _C1)),
            full((1, 2 * _HID)),
            full((1, 2 * _HID)),
            full((9, _HID)),
            full((1, _HID)),
            full((1, _HID)),
            full((2 * _C1, _HID)),
            full((2 * _C1, _HID)),
            full((2 * _C1, _W)),
            full((2 * _C1, _W)),
            full((_W, _L)),
            full((_W, _L)),
            full((_L, _W)),
            full((_L, _W)),
            full((_L, _L)),
            full((_L, _L)),
        ],
        out_specs=x_spec,
        out_shape=jax.ShapeDtypeStruct((_B, _C1, _H, _W), _f32),
    )(x, sc2, w1cc, g1d, b1d, dw9, dsc, d_b.reshape(1, _HID), w2e, w2o, g2m,
      be, jnp.asarray(_S0), jnp.asarray(_S1), jnp.asarray(_S0.T),
      jnp.asarray(_S1.T), jnp.asarray(_RP), jnp.asarray(_RM))
    return delta


# speedup vs baseline: 1.0134x; 1.0134x over previous
import math

import jax
import jax.numpy as jnp
import numpy as np
from jax.experimental import pallas as pl

_B, _C1, _H, _W = 4, 192, 224, 224
_P = 4
_GH, _GW = _H // _P, _W // _P
_NP = _GH * _GW
_K = min(_NP, max(1, int(math.ceil(_NP * 0.25))))
_HID = 96
_RHID = 48
_EPS = 1e-5
_ROWS = 2
_BY = _ROWS * _P
_NB = _GH // _ROWS
_L = 2 * _GW

_f32 = jnp.float32


def _build_consts():
    s0 = np.zeros((_W, _L), np.float32)
    s1 = np.zeros((_W, _L), np.float32)
    for p in range(_GW):
        for ox in range(2):
            s0[4 * p + 2 * ox + 0, 2 * p + ox] = 1.0
            s1[4 * p + 2 * ox + 1, 2 * p + ox] = 1.0
    rp = np.zeros((_L, _L), np.float32)
    rm = np.zeros((_L, _L), np.float32)
    for j in range(_GW):
        rp[2 * j, 2 * j + 1] = 1.0
        rm[2 * j + 1, 2 * j] = 1.0
    spool = np.zeros((_W, _GW), np.float32)
    for p in range(_GW):
        for px in range(4):
            spool[4 * p + px, p] = 1.0
    return s0, s1, rp, rm, spool


_S0, _S1, _RP, _RM, _SPOOL = _build_consts()


def _dotx(a, b):
    return jnp.dot(a, b, precision=jax.lax.Precision.HIGHEST,
                   preferred_element_type=_f32)


def _dotd(a, b):
    return jnp.dot(a.astype(jnp.bfloat16), b.astype(jnp.bfloat16),
                   preferred_element_type=_f32)


def _silu(v):
    return v * jax.nn.sigmoid(v)


def _pool_route_body(x_ref, spool_ref, w1_ref, g1_ref, b1_ref, w2_ref,
                     b2_ref, out_ref):
    xb = x_ref[0]
    spool = spool_ref[...]
    rows = []
    for r2 in range(_ROWS):
        xs = jax.lax.slice(xb, (0, r2 * _P, 0), (_C1, r2 * _P + _P, _W))
        ysum = jnp.sum(xs, axis=1)
        pooled = _dotx(ysum, spool) * _f32(1.0 / 16.0)
        s1 = _dotd(w1_ref[...], pooled)
        s1 = s1 * g1_ref[0][:, None] + b1_ref[0][:, None]
        s1 = _silu(s1)
        lg = _dotd(w2_ref[...], s1)
        rows.append(lg + b2_ref[0][:, None])
    out_ref[0, 0] = jnp.concatenate(rows, axis=0)


def _threshold_body(lg_ref, out_ref):
    s = lg_ref[...]
    lo = jnp.min(s, axis=1, keepdims=True) - 1.0
    hi = jnp.max(s, axis=1, keepdims=True) + 1.0
    kf = _f32(_K)

    def step(_, carry):
        lo, hi = carry
        mid = (lo + hi) * 0.5
        cnt = jnp.sum((s >= mid).astype(_f32), axis=1, keepdims=True)
        pred = cnt >= kf
        return jnp.where(pred, mid, lo), jnp.where(pred, hi, mid)

    lo, hi = jax.lax.fori_loop(0, 60, step, (lo, hi))
    mask = (s >= lo).astype(_f32)
    out_ref[...] = jax.nn.sigmoid(s) * mask


def _mov(a, b):
    hi = a.astype(jnp.bfloat16)
    lo = (a - hi.astype(_f32)).astype(jnp.bfloat16)
    bb = b.astype(jnp.bfloat16)
    return (jnp.dot(hi, bb, preferred_element_type=_f32) +
            jnp.dot(lo, bb, preferred_element_type=_f32))


def _expert_body(x_ref, sc_ref, w1cc_ref, g1d_ref, b1d_ref, dw9_ref, dg_ref,
                 db_ref, w2e_ref, w2o_ref, g2m_ref, be_ref, s0_ref, s1_ref,
                 s0t_ref, s1t_ref, rp_ref, rm_ref, out_ref):
    xb = x_ref[0]
    s0 = s0_ref[...]
    s1 = s1_ref[...]
    s0t = s0t_ref[...]
    s1t = s1t_ref[...]
    rp = rp_ref[...]
    rm = rm_ref[...]
    w1cc = w1cc_ref[...]
    w2e = w2e_ref[...]
    w2o = w2o_ref[...]
    g2m = g2m_ref[...]
    be = be_ref[...]
    g1d = g1d_ref[0][:, None]
    b1d = b1d_ref[0][:, None]
    dg = dg_ref[0][:, None]
    db = db_ref[0][:, None]
    for r2 in range(_ROWS):
        xs = jax.lax.slice(xb, (0, r2 * _P, 0), (_C1, r2 * _P + _P, _W))
        sc = sc_ref[0, 0, r2]
        gs = []
        for oy in range(2):
            y2 = jnp.concatenate(
                [xs[:, 2 * oy, :], xs[:, 2 * oy + 1, :]], axis=0)
            gs.append(_dotd(w1cc, y2))
        ga = jnp.concatenate([gs[0][:_HID], gs[1][:_HID]], axis=0)
        gb = jnp.concatenate([gs[0][_HID:], gs[1][_HID:]], axis=0)
        tl = _mov(ga, s0) + _mov(gb, s1)
        tl = _silu(tl * g1d + b1d)
        tp = _mov(tl, rp)
        tm = _mov(tl, rm)
        us = []
        for oy_ in range(2):
            acc = None
            for oy in range(2):
                ky = 1 + oy - oy_
                k0 = dw9_ref[ky * 3 + 0][:, None]
                k1 = dw9_ref[ky * 3 + 1][:, None]
                k2 = dw9_ref[ky * 3 + 2][:, None]
                sl0, sl1 = oy * _HID, (oy + 1) * _HID
                term = (tl[sl0:sl1] * k1 + tp[sl0:sl1] * k0 +
                        tm[sl0:sl1] * k2)
                acc = term if acc is None else acc + term
            us.append(_silu(acc * dg + db))
        u2 = jnp.concatenate(us, axis=0)
        ue0 = _mov(u2, s0t)
        ue1 = _mov(u2, s1t)
        for oy_ in range(2):
            sl0, sl1 = oy_ * _HID, (oy_ + 1) * _HID
            r = (_dotd(w2e, ue0[sl0:sl1]) +
                 _dotd(w2o, ue1[sl0:sl1]))
            r = (r * g2m + be) * sc[None, :]
            out_ref[0, :, r2 * _P + oy_ * 2 + 0, :] = r[:_C1]
            out_ref[0, :, r2 * _P + oy_ * 2 + 1, :] = r[_C1:]


@jax.jit
def kernel(x, rw1, r_g, r_b, rw2, rb2, ew1, e1_g, e1_b, dw, d_g, d_b, ew2,
           e2_g, e2_b):
    sq = jnp.sqrt(_f32(1.0) + _f32(_EPS))
    r_gs = (r_g / sq).reshape(1, _RHID)
    e1sc = (e1_g / sq).reshape(1, _HID)
    dsc = (d_g / sq).reshape(1, _HID)
    e2v = (e2_g / sq)

    rw1r = rw1.reshape(_RHID, _C1)
    rw2r = rw2.reshape(1, _RHID)
    ew1r = ew1.reshape(_HID, 4 * _C1)
    ew2r = ew2.reshape(4 * _C1, _HID)

    w1r = ew1r.reshape(_HID, _C1, 2, 2)
    w1c0 = w1r[:, :, :, 0].transpose(0, 2, 1).reshape(_HID, 2 * _C1)
    w1c1 = w1r[:, :, :, 1].transpose(0, 2, 1).reshape(_HID, 2 * _C1)
    w1cc = jnp.concatenate([w1c0, w1c1], axis=0)
    w2r = ew2r.reshape(_C1, 2, 2, _HID)
    w2e = w2r[:, :, 0, :].transpose(1, 0, 2).reshape(2 * _C1, _HID)
    w2o = w2r[:, :, 1, :].transpose(1, 0, 2).reshape(2 * _C1, _HID)
    lane_dx = (np.arange(_W) % 2).astype(np.float32)
    m_e = jnp.asarray((1.0 - lane_dx)[None, :])
    m_o = jnp.asarray(lane_dx[None, :])
    ev = e2v.reshape(_C1, 2, 2)
    eb = e2_b.reshape(_C1, 2, 2)
    col = lambda a, dx: a[:, :, dx].transpose(1, 0).reshape(2 * _C1, 1)
    g2m = col(ev, 0) * m_e + col(ev, 1) * m_o
    be = col(eb, 0) * m_e + col(eb, 1) * m_o
    g1d = jnp.concatenate([e1sc, e1sc], axis=1)
    b1d = jnp.concatenate([e1_b.reshape(1, _HID)] * 2, axis=1)
    dw9 = dw.reshape(_HID, 3, 3).transpose(1, 2, 0).reshape(9, _HID)

    grid = (_B, _NB)
    x_spec = pl.BlockSpec((1, _C1, _BY, _W), lambda b, r: (b, 0, r, 0))
    lg_spec = pl.BlockSpec((1, 1, _ROWS, _GW), lambda b, r: (b, r, 0, 0))
    sc_spec = pl.BlockSpec((1, 1, _ROWS, _W), lambda b, r: (b, r, 0, 0))
    full = lambda shape: pl.BlockSpec(shape, lambda b, r: (0,) * len(shape))

    logits = pl.pallas_call(
        _pool_route_body,
        grid=grid,
        in_specs=[
            x_spec,
            full((_W, _GW)),
            full((_RHID, _C1)),
            full((1, _RHID)),
            full((1, _RHID)),
            full((1, _RHID)),
            full((1, 1)),
        ],
        out_specs=lg_spec,
        out_shape=jax.ShapeDtypeStruct((_B, _NB, _ROWS, _GW), _f32),
    )(x, jnp.asarray(_SPOOL), rw1r, r_gs, r_b.reshape(1, _RHID), rw2r,
      rb2.reshape(1, 1))

    scale = pl.pallas_call(
        _threshold_body,
        out_shape=jax.ShapeDtypeStruct((_B, _NP), _f32),
    )(logits.reshape(_B, _NP))
    sc2 = jnp.repeat(scale.reshape(_B, _GH, _GW), 4, axis=2)
    sc2 = sc2.reshape(_B, _NB, _ROWS, _W)

    delta = pl.pallas_call(
        _expert_body,
        grid=grid,
        in_specs=[
            x_spec,
            sc_spec,
            full((2 * _HID, 2 * _C1)),
            full((1, 2 * _HID)),
            full((1, 2 * _HID)),
            full((9, _HID)),
            full((1, _HID)),
            full((1, _HID)),
            full((2 * _C1, _HID)),
            full((2 * _C1, _HID)),
            full((2 * _C1, _W)),
            full((2 * _C1, _W)),
            full((_W, _L)),
            full((_W, _L)),
            full((_L, _W)),
            full((_L, _W)),
            full((_L, _L)),
            full((_L, _L)),
        ],
        out_specs=x_spec,
        out_shape=jax.ShapeDtypeStruct((_B, _C1, _H, _W), _f32),
    )(x, sc2, w1cc, g1d, b1d, dw9, dsc, d_b.reshape(1, _HID), w2e, w2o, g2m,
      be, jnp.asarray(_S0), jnp.asarray(_S1), jnp.asarray(_S0.T),
      jnp.asarray(_S1.T), jnp.asarray(_RP), jnp.asarray(_RM))
    return delta
